# baseline (speedup 1.0000x reference)
import functools

import jax
import jax.numpy as jnp
from jax import lax
from jax.experimental import pallas as pl
from jax.experimental.pallas import tpu as pltpu
from jax.experimental.pallas import tpu_sc as plsc

_P = (100, 100, 100)
_BB = 512
_NC = 2
_NS = 16
_NW = _NC * _NS
_CH = 64
_I2P = 104
_TB = 4


def _table_kernel(c1t_ref, c2q_ref, rl_ref, mask2_ref, out_ref):
    for b in range(_TB):
        mt = c1t_ref[b]
        w1 = jnp.dot(mt, rl_ref[...], preferred_element_type=jnp.float32)
        w4 = jnp.concatenate([w1, w1, w1, w1], axis=0) * mask2_ref[...]
        out_ref[b] = jnp.dot(c2q_ref[...], w4, preferred_element_type=jnp.float32)


def _build_table(c1t, c2q, rl, mask2):
    return pl.pallas_call(
        _table_kernel,
        grid=(_P[1] // _TB,),
        in_specs=[
            pl.BlockSpec((_TB, 32, 128), lambda i: (i, 0, 0)),
            pl.BlockSpec((_I2P, 128), lambda i: (0, 0)),
            pl.BlockSpec((128, 512), lambda i: (0, 0)),
            pl.BlockSpec((128, 512), lambda i: (0, 0)),
        ],
        out_specs=pl.BlockSpec((_TB, _I2P, 512), lambda i: (i, 0, 0)),
        out_shape=jax.ShapeDtypeStruct((_P[1], _I2P, 512), jnp.float32),
    )(c1t, c2q, rl, mask2)


def _sc_gather(table, idx):
    B = idx.shape[0]
    bpw = B // _NW
    n_ch = bpw // _CH
    mesh = plsc.VectorSubcoreMesh(core_axis_name="c", subcore_axis_name="s")

    @functools.partial(
        pl.kernel,
        mesh=mesh,
        out_type=jax.ShapeDtypeStruct((B, 512), jnp.float32),
        scratch_types=[
            pltpu.VMEM((n_ch, _CH), jnp.int32),
            pltpu.VMEM((_CH, 512), jnp.float32),
            pltpu.VMEM((_CH, 512), jnp.float32),
            pltpu.SemaphoreType.DMA,
            pltpu.SemaphoreType.DMA,
        ],
    )
    def k(table_hbm, idx_hbm, out_hbm, key_v, rows0, rows1, sem0, sem1):
        wid = lax.axis_index("s") * _NC + lax.axis_index("c")
        base = wid * bpw
        for ch in range(n_ch):
            pltpu.sync_copy(idx_hbm.at[pl.ds(base + ch * _CH, _CH)], key_v.at[ch])
        bufs = (rows0, rows1)
        sems = (sem0, sem1)
        copies = [None, None]
        for ch in range(n_ch):
            copies[ch % 2] = pltpu.async_copy(
                table_hbm.at[key_v.at[ch]], bufs[ch % 2], sems[ch % 2])
            if ch >= 1:
                pb = (ch - 1) % 2
                copies[pb].wait()
                pltpu.sync_copy(
                    bufs[pb], out_hbm.at[pl.ds(base + (ch - 1) * _CH, _CH)])
        pb = (n_ch - 1) % 2
        copies[pb].wait()
        pltpu.sync_copy(bufs[pb], out_hbm.at[pl.ds(base + (n_ch - 1) * _CH, _CH)])

    return k(table, idx)


def _contract_kernel(idx_ref, c0pt_ref, tg_ref, out_ref):
    idx = idx_ref[0, 0, :]
    i0 = idx // (_P[1] * _P[2])
    iota_s = lax.broadcasted_iota(jnp.int32, (128, _BB), 0)
    oh0t = (i0[None, :] == iota_s).astype(jnp.float32)
    a_t = jnp.dot(c0pt_ref[...], oh0t, preferred_element_type=jnp.float32)
    tg_t = tg_ref[...].T
    outs = []
    for q0 in range(4):
        acc = jnp.zeros((16, _BB), jnp.float32)
        for r1 in range(32):
            arow = a_t[q0 * 32 + r1]
            acc = acc + arow[None, :] * tg_t[r1 * 16:(r1 + 1) * 16]
        outs.append(acc)
    out_ref[...] = jnp.concatenate(outs, axis=0).T


def _contract(idx3, c0p, tg):
    B = tg.shape[0]
    grid = B // _BB
    return pl.pallas_call(
        _contract_kernel,
        grid=(grid,),
        in_specs=[
            pl.BlockSpec((1, 1, _BB), lambda i: (i, 0, 0)),
            pl.BlockSpec((128, 128), lambda i: (0, 0)),
            pl.BlockSpec((_BB, 512), lambda i: (i, 0)),
        ],
        out_specs=pl.BlockSpec((_BB, 64), lambda i: (i, 0)),
        out_shape=jax.ShapeDtypeStruct((B, 64), jnp.float32),
    )(idx3, c0p, tg)


@jax.jit
def kernel(indices, core0, core1, core2):
    B = indices.shape[0]
    idx = indices.astype(jnp.int32)
    c1t = core1.reshape(_P[1], 128, 32).transpose(0, 2, 1)
    c2q = core2.reshape(_P[2], 32, 4).transpose(0, 2, 1).reshape(_P[2], 128)
    c2q = jnp.pad(c2q, ((0, _I2P - _P[2]), (0, 0)))
    c0pt = jnp.pad(core0, ((0, 28), (0, 0))).T
    s128 = jnp.arange(128, dtype=jnp.int32)
    l512 = jnp.arange(512, dtype=jnp.int32)
    rl = (s128[:, None] == l512[None, :] // 4).astype(jnp.float32)
    mask2 = (s128[:, None] // 32 == l512[None, :] % 4).astype(jnp.float32)

    table = _build_table(c1t, c2q, rl, mask2).reshape(_P[1] * _I2P, 512)
    pair = ((idx // _P[2]) % _P[1]) * _I2P + idx % _P[2]
    h = B // 2
    outs = []
    for s in range(2):
        pair_h = pair[s * h:(s + 1) * h]
        idx_h = idx[s * h:(s + 1) * h]
        tg = _sc_gather(table, pair_h)
        outs.append(_contract(idx_h.reshape(h // _BB, 1, _BB), c0pt, tg))
    return jnp.concatenate(outs, axis=0)


# speedup vs baseline: 1.0110x; 1.0110x over previous
import functools

import jax
import jax.numpy as jnp
from jax import lax
from jax.experimental import pallas as pl
from jax.experimental.pallas import tpu as pltpu
from jax.experimental.pallas import tpu_sc as plsc

_P = (100, 100, 100)
_BB = 512
_NC = 2
_NS = 16
_NW = _NC * _NS
_CH = 128
_I2P = 104
_TB = 4


def _table_kernel(c1t_ref, c2q_ref, rl_ref, mask2_ref, out_ref):
    for b in range(_TB):
        mt = c1t_ref[b]
        w1 = jnp.dot(mt, rl_ref[...], preferred_element_type=jnp.float32)
        w4 = jnp.concatenate([w1, w1, w1, w1], axis=0) * mask2_ref[...]
        out_ref[b] = jnp.dot(c2q_ref[...], w4, preferred_element_type=jnp.float32)


def _build_table(c1t, c2q, rl, mask2):
    return pl.pallas_call(
        _table_kernel,
        grid=(_P[1] // _TB,),
        in_specs=[
            pl.BlockSpec((_TB, 32, 128), lambda i: (i, 0, 0)),
            pl.BlockSpec((_I2P, 128), lambda i: (0, 0)),
            pl.BlockSpec((128, 512), lambda i: (0, 0)),
            pl.BlockSpec((128, 512), lambda i: (0, 0)),
        ],
        out_specs=pl.BlockSpec((_TB, _I2P, 512), lambda i: (i, 0, 0)),
        out_shape=jax.ShapeDtypeStruct((_P[1], _I2P, 512), jnp.float32),
    )(c1t, c2q, rl, mask2)


def _sc_gather(table, idx):
    B = idx.shape[0]
    bpw = B // _NW
    n_ch = bpw // _CH
    mesh = plsc.VectorSubcoreMesh(core_axis_name="c", subcore_axis_name="s")

    @functools.partial(
        pl.kernel,
        mesh=mesh,
        out_type=jax.ShapeDtypeStruct((B, 512), jnp.float32),
        scratch_types=[
            pltpu.VMEM((n_ch, _CH), jnp.int32),
            pltpu.VMEM((_CH, 512), jnp.float32),
            pltpu.SemaphoreType.DMA,
        ],
    )
    def k(table_hbm, idx_hbm, out_hbm, key_v, rows_v, sem):
        wid = lax.axis_index("s") * _NC + lax.axis_index("c")
        base = wid * bpw
        for ch in range(n_ch):
            pltpu.sync_copy(idx_hbm.at[pl.ds(base + ch * _CH, _CH)], key_v.at[ch])
        for ch in range(n_ch):
            pltpu.async_copy(table_hbm.at[key_v.at[ch]], rows_v, sem).wait()
            pltpu.sync_copy(rows_v, out_hbm.at[pl.ds(base + ch * _CH, _CH)])

    return k(table, idx)


def _contract_kernel(idx_ref, c0pt_ref, tg_ref, out_ref):
    idx = idx_ref[0, 0, :]
    i0 = idx // (_P[1] * _P[2])
    iota_s = lax.broadcasted_iota(jnp.int32, (128, _BB), 0)
    oh0t = (i0[None, :] == iota_s).astype(jnp.float32)
    a_t = jnp.dot(c0pt_ref[...], oh0t, preferred_element_type=jnp.float32)
    tg_t = tg_ref[...].T
    outs = []
    for q0 in range(4):
        acc = jnp.zeros((16, _BB), jnp.float32)
        for r1 in range(32):
            arow = a_t[q0 * 32 + r1]
            acc = acc + arow[None, :] * tg_t[r1 * 16:(r1 + 1) * 16]
        outs.append(acc)
    out_ref[...] = jnp.concatenate(outs, axis=0).T


def _contract(idx3, c0p, tg):
    B = tg.shape[0]
    grid = B // _BB
    return pl.pallas_call(
        _contract_kernel,
        grid=(grid,),
        in_specs=[
            pl.BlockSpec((1, 1, _BB), lambda i: (i, 0, 0)),
            pl.BlockSpec((128, 128), lambda i: (0, 0)),
            pl.BlockSpec((_BB, 512), lambda i: (i, 0)),
        ],
        out_specs=pl.BlockSpec((_BB, 64), lambda i: (i, 0)),
        out_shape=jax.ShapeDtypeStruct((B, 64), jnp.float32),
    )(idx3, c0p, tg)


@jax.jit
def kernel(indices, core0, core1, core2):
    B = indices.shape[0]
    idx = indices.astype(jnp.int32)
    c1t = core1.reshape(_P[1], 128, 32).transpose(0, 2, 1)
    c2q = core2.reshape(_P[2], 32, 4).transpose(0, 2, 1).reshape(_P[2], 128)
    c2q = jnp.pad(c2q, ((0, _I2P - _P[2]), (0, 0)))
    c0pt = jnp.pad(core0, ((0, 28), (0, 0))).T
    s128 = jnp.arange(128, dtype=jnp.int32)
    l512 = jnp.arange(512, dtype=jnp.int32)
    rl = (s128[:, None] == l512[None, :] // 4).astype(jnp.float32)
    mask2 = (s128[:, None] // 32 == l512[None, :] % 4).astype(jnp.float32)

    table = _build_table(c1t, c2q, rl, mask2).reshape(_P[1] * _I2P, 512)
    pair = ((idx // _P[2]) % _P[1]) * _I2P + idx % _P[2]
    tg = _sc_gather(table, pair)
    return _contract(idx.reshape(B // _BB, 1, _BB), c0pt, tg)


# speedup vs baseline: 1.1515x; 1.1391x over previous
import functools

import jax
import jax.numpy as jnp
from jax import lax
from jax.experimental import pallas as pl
from jax.experimental.pallas import tpu as pltpu
from jax.experimental.pallas import tpu_sc as plsc

_P = (100, 100, 100)
_BB = 512
_NC = 2
_NS = 16
_NW = _NC * _NS
_CH = 128
_I2P = 104
_TB = 20


def _table_kernel(c1t_ref, c2q_ref, rl_ref, mask2_ref, out_ref):
    for g in range(_TB // 4):
        mtcat = c1t_ref[...][g * 4:(g + 1) * 4].reshape(128, 128)
        w1cat = jnp.dot(mtcat, rl_ref[...], preferred_element_type=jnp.float32)
        for b in range(4):
            w1 = w1cat[b * 32:(b + 1) * 32]
            w4 = jnp.concatenate([w1, w1, w1, w1], axis=0) * mask2_ref[...]
            out_ref[g * 4 + b] = jnp.dot(
                c2q_ref[...], w4, preferred_element_type=jnp.float32)


def _build_table(c1t, c2q, rl, mask2):
    return pl.pallas_call(
        _table_kernel,
        grid=(_P[1] // _TB,),
        in_specs=[
            pl.BlockSpec((_TB, 32, 128), lambda i: (i, 0, 0)),
            pl.BlockSpec((_I2P, 128), lambda i: (0, 0)),
            pl.BlockSpec((128, 512), lambda i: (0, 0)),
            pl.BlockSpec((128, 512), lambda i: (0, 0)),
        ],
        out_specs=pl.BlockSpec((_TB, _I2P, 512), lambda i: (i, 0, 0)),
        out_shape=jax.ShapeDtypeStruct((_P[1], _I2P, 512), jnp.float32),
    )(c1t, c2q, rl, mask2)


def _sc_gather(table, idx):
    B = idx.shape[0]
    bpw = B // _NW
    n_ch = bpw // _CH
    mesh = plsc.VectorSubcoreMesh(core_axis_name="c", subcore_axis_name="s")

    @functools.partial(
        pl.kernel,
        mesh=mesh,
        out_type=jax.ShapeDtypeStruct((B, 512), jnp.float32),
        scratch_types=[
            pltpu.VMEM((n_ch, _CH), jnp.int32),
            pltpu.VMEM((_CH, 512), jnp.float32),
            pltpu.SemaphoreType.DMA,
        ],
    )
    def k(table_hbm, idx_hbm, out_hbm, key_v, rows_v, sem):
        wid = lax.axis_index("s") * _NC + lax.axis_index("c")
        base = wid * bpw
        for ch in range(n_ch):
            pltpu.sync_copy(idx_hbm.at[pl.ds(base + ch * _CH, _CH)], key_v.at[ch])
        for ch in range(n_ch):
            pltpu.async_copy(table_hbm.at[key_v.at[ch]], rows_v, sem).wait()
            pltpu.sync_copy(rows_v, out_hbm.at[pl.ds(base + ch * _CH, _CH)])

    return k(table, idx)


def _contract_kernel(idx_ref, c0pt_ref, tg_ref, out_ref):
    idx = idx_ref[0, 0, :]
    i0 = idx // (_P[1] * _P[2])
    iota_s = lax.broadcasted_iota(jnp.int32, (128, _BB), 0)
    oh0t = (i0[None, :] == iota_s).astype(jnp.float32)
    a_t = jnp.dot(c0pt_ref[...], oh0t, preferred_element_type=jnp.float32)
    tg_t = tg_ref[...].T
    outs = []
    for q0 in range(4):
        acc = jnp.zeros((16, _BB), jnp.float32)
        for r1 in range(32):
            arow = a_t[q0 * 32 + r1]
            acc = acc + arow[None, :] * tg_t[r1 * 16:(r1 + 1) * 16]
        outs.append(acc)
    out_ref[...] = jnp.concatenate(outs, axis=0).T


def _contract(idx3, c0p, tg):
    B = tg.shape[0]
    grid = B // _BB
    return pl.pallas_call(
        _contract_kernel,
        grid=(grid,),
        in_specs=[
            pl.BlockSpec((1, 1, _BB), lambda i: (i, 0, 0)),
            pl.BlockSpec((128, 128), lambda i: (0, 0)),
            pl.BlockSpec((_BB, 512), lambda i: (i, 0)),
        ],
        out_specs=pl.BlockSpec((_BB, 64), lambda i: (i, 0)),
        out_shape=jax.ShapeDtypeStruct((B, 64), jnp.float32),
    )(idx3, c0p, tg)


@jax.jit
def kernel(indices, core0, core1, core2):
    B = indices.shape[0]
    idx = indices.astype(jnp.int32)
    c1t = core1.reshape(_P[1], 128, 32).transpose(0, 2, 1)
    c2q = core2.reshape(_P[2], 32, 4).transpose(0, 2, 1).reshape(_P[2], 128)
    c2q = jnp.pad(c2q, ((0, _I2P - _P[2]), (0, 0)))
    c0pt = jnp.pad(core0, ((0, 28), (0, 0))).T
    s128 = jnp.arange(128, dtype=jnp.int32)
    l512 = jnp.arange(512, dtype=jnp.int32)
    rl = (s128[:, None] == l512[None, :] // 4).astype(jnp.float32)
    mask2 = (s128[:, None] // 32 == l512[None, :] % 4).astype(jnp.float32)

    table = _build_table(c1t, c2q, rl, mask2).reshape(_P[1] * _I2P, 512)
    pair = ((idx // _P[2]) % _P[1]) * _I2P + idx % _P[2]
    h = B // 2
    outs = []
    for s in range(2):
        pair_h = pair[s * h:(s + 1) * h]
        idx_h = idx[s * h:(s + 1) * h]
        tg = _sc_gather(table, pair_h)
        outs.append(_contract(idx_h.reshape(h // _BB, 1, _BB), c0pt, tg))
    return jnp.concatenate(outs, axis=0)


# speedup vs baseline: 1.1978x; 1.0402x over previous
import functools

import jax
import jax.numpy as jnp
from jax import lax
from jax.experimental import pallas as pl
from jax.experimental.pallas import tpu as pltpu
from jax.experimental.pallas import tpu_sc as plsc

_P = (100, 100, 100)
_BB = 1024
_NC = 2
_NS = 16
_NW = _NC * _NS
_CH = 128
_I2P = 104
_TB = 20


def _table_kernel(c1t_ref, c2q_ref, rl_ref, mask2_ref, out_ref):
    for g in range(_TB // 4):
        mtcat = c1t_ref[...][g * 4:(g + 1) * 4].reshape(128, 128)
        w1cat = jnp.dot(mtcat, rl_ref[...], preferred_element_type=jnp.float32)
        for b in range(4):
            w1 = w1cat[b * 32:(b + 1) * 32]
            w4 = jnp.concatenate([w1, w1, w1, w1], axis=0) * mask2_ref[...]
            out_ref[g * 4 + b] = jnp.dot(
                c2q_ref[...], w4, preferred_element_type=jnp.float32)


def _build_table(c1t, c2q, rl, mask2):
    return pl.pallas_call(
        _table_kernel,
        grid=(_P[1] // _TB,),
        in_specs=[
            pl.BlockSpec((_TB, 32, 128), lambda i: (i, 0, 0)),
            pl.BlockSpec((_I2P, 128), lambda i: (0, 0)),
            pl.BlockSpec((128, 512), lambda i: (0, 0)),
            pl.BlockSpec((128, 512), lambda i: (0, 0)),
        ],
        out_specs=pl.BlockSpec((_TB, _I2P, 512), lambda i: (i, 0, 0)),
        out_shape=jax.ShapeDtypeStruct((_P[1], _I2P, 512), jnp.float32),
    )(c1t, c2q, rl, mask2)


def _sc_gather(table, idx):
    B = idx.shape[0]
    bpw = B // _NW
    n_ch = bpw // _CH
    mesh = plsc.VectorSubcoreMesh(core_axis_name="c", subcore_axis_name="s")

    @functools.partial(
        pl.kernel,
        mesh=mesh,
        out_type=jax.ShapeDtypeStruct((B, 512), jnp.float32),
        scratch_types=[
            pltpu.VMEM((n_ch, _CH), jnp.int32),
            pltpu.VMEM((_CH, 512), jnp.float32),
            pltpu.SemaphoreType.DMA,
        ],
    )
    def k(table_hbm, idx_hbm, out_hbm, key_v, rows_v, sem):
        wid = lax.axis_index("s") * _NC + lax.axis_index("c")
        base = wid * bpw
        for ch in range(n_ch):
            pltpu.sync_copy(idx_hbm.at[pl.ds(base + ch * _CH, _CH)], key_v.at[ch])
        for ch in range(n_ch):
            pltpu.async_copy(table_hbm.at[key_v.at[ch]], rows_v, sem).wait()
            pltpu.sync_copy(rows_v, out_hbm.at[pl.ds(base + ch * _CH, _CH)])

    return k(table, idx)


def _contract_kernel(idx_ref, c0pt_ref, tg_ref, out_ref):
    idx = idx_ref[0, 0, :]
    i0 = idx // (_P[1] * _P[2])
    iota_s = lax.broadcasted_iota(jnp.int32, (128, _BB), 0)
    oh0t = (i0[None, :] == iota_s).astype(jnp.float32)
    a_t = jnp.dot(c0pt_ref[...], oh0t, preferred_element_type=jnp.float32)
    tg_t = tg_ref[...].T
    outs = []
    for q0 in range(4):
        acc = jnp.zeros((16, _BB), jnp.float32)
        for r1 in range(32):
            arow = a_t[q0 * 32 + r1]
            acc = acc + arow[None, :] * tg_t[r1 * 16:(r1 + 1) * 16]
        outs.append(acc)
    out_ref[...] = jnp.concatenate(outs, axis=0).T


def _contract(idx3, c0p, tg):
    B = tg.shape[0]
    grid = B // _BB
    return pl.pallas_call(
        _contract_kernel,
        grid=(grid,),
        in_specs=[
            pl.BlockSpec((1, 1, _BB), lambda i: (i, 0, 0)),
            pl.BlockSpec((128, 128), lambda i: (0, 0)),
            pl.BlockSpec((_BB, 512), lambda i: (i, 0)),
        ],
        out_specs=pl.BlockSpec((_BB, 64), lambda i: (i, 0)),
        out_shape=jax.ShapeDtypeStruct((B, 64), jnp.float32),
    )(idx3, c0p, tg)


@jax.jit
def kernel(indices, core0, core1, core2):
    B = indices.shape[0]
    idx = indices.astype(jnp.int32)
    c1t = core1.reshape(_P[1], 128, 32).transpose(0, 2, 1)
    c2q = core2.reshape(_P[2], 32, 4).transpose(0, 2, 1).reshape(_P[2], 128)
    c2q = jnp.pad(c2q, ((0, _I2P - _P[2]), (0, 0)))
    c0pt = jnp.pad(core0, ((0, 28), (0, 0))).T
    s128 = jnp.arange(128, dtype=jnp.int32)
    l512 = jnp.arange(512, dtype=jnp.int32)
    rl = (s128[:, None] == l512[None, :] // 4).astype(jnp.float32)
    mask2 = (s128[:, None] // 32 == l512[None, :] % 4).astype(jnp.float32)

    table = _build_table(c1t, c2q, rl, mask2).reshape(_P[1] * _I2P, 512)
    pair = ((idx // _P[2]) % _P[1]) * _I2P + idx % _P[2]
    h = B // 2
    outs = []
    for s in range(2):
        pair_h = pair[s * h:(s + 1) * h]
        idx_h = idx[s * h:(s + 1) * h]
        tg = _sc_gather(table, pair_h)
        outs.append(_contract(idx_h.reshape(h // _BB, 1, _BB), c0pt, tg))
    return jnp.concatenate(outs, axis=0)
